# baseline (speedup 1.0000x reference)
import jax
import jax.numpy as jnp
from jax import lax
from jax.experimental import pallas as pl
from jax.experimental.pallas import tpu as pltpu

B, L = 16, 2048
H = 32
NTOK = B * L

_CL = (((1,), (1,)), ((), ()))
_CS = (((1,), (0,)), ((), ()))


def _dot(x, w, dims):
    return lax.dot_general(x, w, dims, preferred_element_type=jnp.float32)


def _tc_body(xall_ref, len_ref, etbl,
             wp1, bp1, wp2, bp2, wc1, bc1, wc2, bc2,
             w1, b1, w2, b2, out_ref):
    lenv = len_ref[...]
    lenf = lenv.astype(jnp.float32)

    wblk = jnp.concatenate([
        jnp.concatenate([wp1[...], jnp.zeros((H, 2), jnp.float32)], axis=1),
        jnp.concatenate([jnp.zeros((H, 3), jnp.float32), wc1[...]], axis=1),
    ], axis=0)
    bcat = jnp.concatenate([bp1[...], bc1[...]], axis=0).reshape(1, 2 * H)

    xall = xall_ref[...].reshape(NTOK, 12)
    h = jnp.maximum(_dot(xall[:, :5], wblk, _CL) + bcat, 0.0)

    colio = lax.broadcasted_iota(jnp.int32, (B, NTOK), 1)
    rowio = lax.broadcasted_iota(jnp.int32, (B, NTOK), 0)
    tok = jnp.bitwise_and(colio, L - 1)
    sid = lax.shift_right_logical(colio, 11)
    maskM = jnp.where((sid == rowio) & (tok < lenv), 1.0, 0.0)
    S = _dot(maskM, h, _CS)
    s = _dot(maskM, xall[:, 5:12], _CS)
    hps = S[:, :H]
    hcs = S[:, H:]

    cp_pool = _dot(hps / lenf, wp2[...], _CL) + bp2[...].reshape(1, H)
    cc_pool = _dot(hcs / lenf, wc2[...], _CL) + bc2[...].reshape(1, H)

    e = etbl[...]
    dEP = jnp.concatenate([e[1:2] - e[0:1], e[3:4] - e[2:3], e[5:6] - e[4:5],
                           e[7:8] - e[6:7], e[9:10] - e[8:9]], axis=0)
    e0p = e[0:1] + e[2:3] + e[4:5] + e[6:7] + e[8:9]
    dEC = jnp.concatenate([e[11:12] - e[10:11], e[13:14] - e[12:13]], axis=0)
    e0c = e[10:11] + e[12:13]

    catp_pool = (e0p * lenf + _dot(s[:, :5], dEP, _CS)) / (5.0 * lenf)
    catc_pool = (e0c * lenf + _dot(s[:, 5:7], dEC, _CS)) / (2.0 * lenf)

    pooled = jnp.concatenate([catp_pool, catc_pool, cp_pool, cc_pool], axis=1)
    hh = jnp.maximum(_dot(pooled, w1[...], _CL) + b1[...].reshape(1, 64), 0.0)
    out_ref[...] = jnp.maximum(_dot(hh, w2[...], _CL) + b2[...].reshape(1, 2), 0.0)


def kernel(cont_p, cont_c, cat_p, cat_c, lengths,
           Wp1, bp1, Wp2, bp2, Wc1, bc1, Wc2, bc2,
           Eg, Ek, Epr, Ej, Er, Epl, Ea,
           W1, b1, W2, b2):
    f32 = jnp.float32
    xall = jnp.concatenate([
        cont_p, cont_c,
        cat_p[:, :, :4].astype(f32), cat_c[:, :, 0:1].astype(f32),
        cat_c[:, :, 1:2].astype(f32), cat_p[:, :, 4:5].astype(f32),
    ], axis=2)
    etbl = jnp.concatenate([Eg[:2], Ek[:2], Epr[:2], Ej[:2], Epl[:2],
                            Ea[:2], Er[:2]], axis=0)

    full = lambda shape: pl.BlockSpec(shape, lambda: (0,) * len(shape))
    out = pl.pallas_call(
        _tc_body,
        in_specs=[
            full((B, L, 12)), full((B, 1)), full((14, H)),
            full((H, 3)), full((H,)), full((H, H)), full((H,)),
            full((H, 2)), full((H,)), full((H, H)), full((H,)),
            full((64, 128)), full((64,)), full((2, 64)), full((2,)),
        ],
        out_specs=full((B, 2)),
        out_shape=jax.ShapeDtypeStruct((B, 2), jnp.float32),
    )(xall, lengths.reshape(B, 1), etbl,
      Wp1, bp1, Wp2, bp2, Wc1, bc1, Wc2, bc2, W1, b1, W2, b2)
    return out


# speedup vs baseline: 8.4114x; 8.4114x over previous
import jax
import jax.numpy as jnp
from jax import lax
from jax.experimental import pallas as pl
from jax.experimental.pallas import tpu as pltpu

B, L = 16, 2048
H = 32
NTOK = B * L

_CL = (((1,), (1,)), ((), ()))
_CS = (((1,), (0,)), ((), ()))


def _dot(x, w, dims):
    return lax.dot_general(x, w, dims, preferred_element_type=jnp.float32)


def _tc_body(xfm_ref, len_ref, etbl,
             wp1, bp1, wp2, bp2, wc1, bc1, wc2, bc2,
             w1, b1, w2, b2, out_ref):
    lenv = len_ref[...]
    lenf = lenv.astype(jnp.float32)

    tokio = lax.broadcasted_iota(jnp.int32, (B, L), 1)
    maskF = jnp.where(tokio < lenv, 1.0, 0.0)

    wblk = jnp.concatenate([
        jnp.concatenate([wp1[...], jnp.zeros((H, 2), jnp.float32)], axis=1),
        jnp.concatenate([jnp.zeros((H, 3), jnp.float32), wc1[...]], axis=1),
    ], axis=0)
    bcat = jnp.concatenate([bp1[...], bc1[...]], axis=0).reshape(2 * H, 1)

    xfm = xfm_ref[...]
    hh = jnp.maximum(_dot(wblk, xfm[0:5, :], _CS) + bcat, 0.0)

    srows = []
    hrows = []
    for b in range(B):
        mb = maskF[b:b + 1, :]
        hb = hh[:, b * L:(b + 1) * L] * mb
        hrows.append(jnp.sum(hb, axis=1).reshape(1, 2 * H))
        cb = xfm[5:12, b * L:(b + 1) * L] * mb
        srows.append(jnp.sum(cb, axis=1).reshape(1, 7))
    S = jnp.concatenate(hrows, axis=0)
    s = jnp.concatenate(srows, axis=0)

    hps = S[:, :H]
    hcs = S[:, H:]
    cp_pool = _dot(hps / lenf, wp2[...], _CL) + bp2[...].reshape(1, H)
    cc_pool = _dot(hcs / lenf, wc2[...], _CL) + bc2[...].reshape(1, H)

    e = etbl[...]
    dEP = jnp.concatenate([e[1:2] - e[0:1], e[3:4] - e[2:3], e[5:6] - e[4:5],
                           e[7:8] - e[6:7], e[9:10] - e[8:9]], axis=0)
    e0p = e[0:1] + e[2:3] + e[4:5] + e[6:7] + e[8:9]
    dEC = jnp.concatenate([e[11:12] - e[10:11], e[13:14] - e[12:13]], axis=0)
    e0c = e[10:11] + e[12:13]

    catp_pool = (e0p * lenf + _dot(s[:, :5], dEP, _CS)) / (5.0 * lenf)
    catc_pool = (e0c * lenf + _dot(s[:, 5:7], dEC, _CS)) / (2.0 * lenf)

    pooled = jnp.concatenate([catp_pool, catc_pool, cp_pool, cc_pool], axis=1)
    hd = jnp.maximum(_dot(pooled, w1[...], _CL) + b1[...].reshape(1, 64), 0.0)
    out_ref[...] = jnp.maximum(_dot(hd, w2[...], _CL) + b2[...].reshape(1, 2), 0.0)


def kernel(cont_p, cont_c, cat_p, cat_c, lengths,
           Wp1, bp1, Wp2, bp2, Wc1, bc1, Wc2, bc2,
           Eg, Ek, Epr, Ej, Er, Epl, Ea,
           W1, b1, W2, b2):
    f32 = jnp.float32
    xfm = jnp.concatenate([
        cont_p, cont_c,
        cat_p[:, :, :4].astype(f32), cat_c[:, :, 0:1].astype(f32),
        cat_c[:, :, 1:2].astype(f32), cat_p[:, :, 4:5].astype(f32),
    ], axis=2).transpose(2, 0, 1).reshape(12, NTOK)
    etbl = jnp.concatenate([Eg[:2], Ek[:2], Epr[:2], Ej[:2], Epl[:2],
                            Ea[:2], Er[:2]], axis=0)

    full = lambda shape: pl.BlockSpec(shape, lambda: (0,) * len(shape))
    out = pl.pallas_call(
        _tc_body,
        in_specs=[
            full((12, NTOK)), full((B, 1)), full((14, H)),
            full((H, 3)), full((H,)), full((H, H)), full((H,)),
            full((H, 2)), full((H,)), full((H, H)), full((H,)),
            full((64, 128)), full((64,)), full((2, 64)), full((2,)),
        ],
        out_specs=full((B, 2)),
        out_shape=jax.ShapeDtypeStruct((B, 2), jnp.float32),
    )(xfm, lengths.reshape(B, 1), etbl,
      Wp1, bp1, Wp2, bp2, Wc1, bc1, Wc2, bc2, W1, b1, W2, b2)
    return out
